# baseline (speedup 1.0000x reference)
import jax
import jax.numpy as jnp
from jax.experimental import pallas as pl
from jax.experimental.pallas import tpu as pltpu

_B, _C, _H, _W = 2, 96, 512, 512
_HO, _WO = 258, 258


def _chmin_kernel(x_ref, o_ref):
    ho = pl.program_id(0)
    hh = jnp.clip(ho - 1, 0, 255)
    base = 2 * (hh % 4)
    xv = x_ref[...].reshape(_B * _C, 8, _W)
    si = jax.lax.broadcasted_iota(jnp.int32, (_B * _C, 8, _W), 1)
    vv = jnp.take_along_axis(xv, base + (si % 4) // 2, axis=1)
    s2v = jax.lax.broadcasted_iota(jnp.int32, (_B * _C, 8, 128), 1) % 2
    jj = jax.lax.broadcasted_iota(jnp.int32, (_B * _C, 8, 128), 2)
    chunks = [vv[:, :, 128 * k : 128 * (k + 1)] for k in range(4)]

    def _wg(k):
        w = 2 * jnp.clip(128 * k + jj - 1, 0, 255) + s2v
        q = w % 128
        return [jnp.take_along_axis(c, q, axis=2) for c in chunks]

    t0 = _wg(0)
    g0 = jnp.where(jj <= 64, t0[0], t0[1])
    t1 = _wg(1)
    g1 = jnp.where(jj == 0, t1[1], jnp.where(jj <= 64, t1[2], t1[3]))
    t2 = _wg(2)
    g2 = t2[3]
    vw = jnp.concatenate([g0, g1, g2], axis=2)
    rr = jax.lax.broadcasted_iota(jnp.int32, (8 * _C, 4 * _C), 0)
    cc = jax.lax.broadcasted_iota(jnp.int32, (8 * _C, 4 * _C), 1)
    E = ((rr % 8 < 4) & (cc == 4 * (rr // 8) + rr % 8)).astype(jnp.bfloat16)
    dn = (((0,), (0,)), ((), ()))
    for b in range(_B):
        Lb = vw[_C * b : _C * (b + 1)].reshape(8 * _C, 384)
        Lh = Lb.astype(jnp.bfloat16)
        Ll = (Lb - Lh.astype(jnp.float32)).astype(jnp.bfloat16)
        outb = jax.lax.dot_general(
            Lh, E, dn, preferred_element_type=jnp.float32
        ) + jax.lax.dot_general(Ll, E, dn, preferred_element_type=jnp.float32)
        o_ref[0, :, b, :] = outb[0:_WO, :]


def kernel(x):
    t = pl.pallas_call(
        _chmin_kernel,
        grid=(_HO,),
        in_specs=[
            pl.BlockSpec(
                (_B, _C, 8, _W),
                lambda ho: (0, 0, jnp.clip(ho - 1, 0, 255) // 4, 0),
            )
        ],
        out_specs=pl.BlockSpec((1, _WO, _B, 4 * _C), lambda ho: (ho, 0, 0, 0)),
        out_shape=jax.ShapeDtypeStruct((_HO, _WO, _B, 4 * _C), jnp.float32),
        compiler_params=pltpu.CompilerParams(
            dimension_semantics=("arbitrary",),
        ),
    )(x)
    return t.transpose(2, 3, 0, 1)
